# initial kernel scaffold (unmeasured)
import jax
import jax.numpy as jnp
from jax import lax
from jax.experimental import pallas as pl
from jax.experimental.pallas import tpu as pltpu

N_DEV = 4
M_PER = 2048
HALF = 1024
TILE = 512
K = 8192
N_SH = 1024


def _gelu(y):
    c = 0.7978845608028654
    return 0.5 * y * (1.0 + jnp.tanh(c * (y + 0.044715 * y * y * y)))


def kernel(x, w_mat):
    x = x.astype(jnp.bfloat16)
    w = w_mat.astype(jnp.bfloat16)

    def body(x_ref, w_ref, out_ref, xg_ref, xtile, otile,
             send_a, recv_a, send_b, recv_b, in_sem, out_sem):
        me = lax.axis_index("i")
        left = lax.rem(me + N_DEV - 1, N_DEV)
        right = lax.rem(me + 1, N_DEV)

        barrier = pltpu.get_barrier_semaphore()
        for nbr in (left, right):
            pl.semaphore_signal(barrier, inc=1, device_id=(nbr,),
                                device_id_type=pl.DeviceIdType.MESH)
        pl.semaphore_wait(barrier, 2)

        def compute_half(origin, s, from_x):
            for t in range(HALF // TILE):
                r = s * HALF + t * TILE
                if from_x:
                    src = x_ref.at[pl.ds(r, TILE), :]
                else:
                    src = xg_ref.at[origin, pl.ds(r, TILE), :]
                cp_in = pltpu.make_async_copy(src, xtile, in_sem)
                cp_in.start()
                cp_in.wait()
                y = jnp.dot(xtile[...], w_ref[...],
                            preferred_element_type=jnp.float32)
                otile[...] = _gelu(y)
                cp_out = pltpu.make_async_copy(
                    otile, out_ref.at[pl.ds(origin * M_PER + r, TILE), :],
                    out_sem)
                cp_out.start()
                cp_out.wait()

        for h in range(1, N_DEV):
            o_a = lax.rem(me + N_DEV - (h - 1), N_DEV)
            o_b = lax.rem(me + (h - 1), N_DEV)
            if h == 1:
                src_a = x_ref.at[pl.ds(0, HALF), :]
                src_b = x_ref.at[pl.ds(HALF, HALF), :]
            else:
                src_a = xg_ref.at[o_a, pl.ds(0, HALF), :]
                src_b = xg_ref.at[o_b, pl.ds(HALF, HALF), :]
            rdma_a = pltpu.make_async_remote_copy(
                src_ref=src_a, dst_ref=xg_ref.at[o_a, pl.ds(0, HALF), :],
                send_sem=send_a.at[h - 1], recv_sem=recv_a.at[h - 1],
                device_id=(right,), device_id_type=pl.DeviceIdType.MESH)
            rdma_b = pltpu.make_async_remote_copy(
                src_ref=src_b, dst_ref=xg_ref.at[o_b, pl.ds(HALF, HALF), :],
                send_sem=send_b.at[h - 1], recv_sem=recv_b.at[h - 1],
                device_id=(left,), device_id_type=pl.DeviceIdType.MESH)
            rdma_a.start()
            rdma_b.start()
            if h == 1:
                compute_half(me, 0, True)
                compute_half(me, 1, True)
            else:
                compute_half(o_a, 0, False)
                compute_half(o_b, 1, False)
            rdma_a.wait()
            rdma_b.wait()

        compute_half(lax.rem(me + 1, N_DEV), 0, False)
        compute_half(lax.rem(me + 3, N_DEV), 1, False)

    out, _ = pl.pallas_call(
        body,
        out_shape=[
            jax.ShapeDtypeStruct((N_DEV * M_PER, N_SH), jnp.float32),
            jax.ShapeDtypeStruct((N_DEV, M_PER, K), jnp.bfloat16),
        ],
        in_specs=[
            pl.BlockSpec(memory_space=pltpu.ANY),
            pl.BlockSpec(memory_space=pltpu.VMEM),
        ],
        out_specs=[
            pl.BlockSpec(memory_space=pltpu.ANY),
            pl.BlockSpec(memory_space=pltpu.ANY),
        ],
        scratch_shapes=[
            pltpu.VMEM((TILE, K), jnp.bfloat16),
            pltpu.VMEM((TILE, N_SH), jnp.float32),
            pltpu.SemaphoreType.DMA((N_DEV - 1,)),
            pltpu.SemaphoreType.DMA((N_DEV - 1,)),
            pltpu.SemaphoreType.DMA((N_DEV - 1,)),
            pltpu.SemaphoreType.DMA((N_DEV - 1,)),
            pltpu.SemaphoreType.DMA,
            pltpu.SemaphoreType.DMA,
        ],
        compiler_params=pltpu.CompilerParams(collective_id=0),
    )(x, w)
    return out


# baseline (device time: 677246 ns/iter reference)
import jax
import jax.numpy as jnp
from jax import lax
from jax.experimental import pallas as pl
from jax.experimental.pallas import tpu as pltpu

N_DEV = 4
M_PER = 2048
HALF = 1024
TILE = 512
K = 8192
N_SH = 1024


def _gelu(y):
    c = 0.7978845608028654
    return 0.5 * y * (1.0 + jnp.tanh(c * (y + 0.044715 * y * y * y)))


def kernel(x, w_mat):
    x = x.astype(jnp.bfloat16)
    w = w_mat.astype(jnp.bfloat16)

    def body(x_ref, w_ref, out_ref, xg_ref, xtile, otile,
             send_a, recv_a, send_b, recv_b, in_sem, out_sem):
        me = lax.axis_index("i")
        left = lax.rem(me + N_DEV - 1, N_DEV)
        right = lax.rem(me + 1, N_DEV)

        barrier = pltpu.get_barrier_semaphore()
        for nbr in (left, right):
            pl.semaphore_signal(barrier, inc=1, device_id=(nbr,),
                                device_id_type=pl.DeviceIdType.MESH)
        pl.semaphore_wait(barrier, 2)

        def compute_halves(o_top, o_bot, from_x):
            def tile_body(t, carry):
                origin = jnp.where(t < 2, o_top, o_bot)
                r = t * TILE
                if from_x:
                    src = x_ref.at[pl.ds(r, TILE), :]
                else:
                    src = xg_ref.at[origin, pl.ds(r, TILE), :]
                cp_in = pltpu.make_async_copy(src, xtile, in_sem)
                cp_in.start()
                cp_in.wait()
                y = jnp.dot(xtile[...], w_ref[...],
                            preferred_element_type=jnp.float32)
                otile[...] = _gelu(y)
                cp_out = pltpu.make_async_copy(
                    otile, out_ref.at[pl.ds(origin * M_PER + r, TILE), :],
                    out_sem)
                cp_out.start()
                cp_out.wait()
                return carry
            lax.fori_loop(0, (2 * HALF) // TILE, tile_body, 0)

        for h in range(1, N_DEV):
            o_a = lax.rem(me + N_DEV - (h - 1), N_DEV)
            o_b = lax.rem(me + (h - 1), N_DEV)
            if h == 1:
                src_a = x_ref.at[pl.ds(0, HALF), :]
                src_b = x_ref.at[pl.ds(HALF, HALF), :]
            else:
                src_a = xg_ref.at[o_a, pl.ds(0, HALF), :]
                src_b = xg_ref.at[o_b, pl.ds(HALF, HALF), :]
            rdma_a = pltpu.make_async_remote_copy(
                src_ref=src_a, dst_ref=xg_ref.at[o_a, pl.ds(0, HALF), :],
                send_sem=send_a.at[h - 1], recv_sem=recv_a.at[h - 1],
                device_id=(right,), device_id_type=pl.DeviceIdType.MESH)
            rdma_b = pltpu.make_async_remote_copy(
                src_ref=src_b, dst_ref=xg_ref.at[o_b, pl.ds(HALF, HALF), :],
                send_sem=send_b.at[h - 1], recv_sem=recv_b.at[h - 1],
                device_id=(left,), device_id_type=pl.DeviceIdType.MESH)
            rdma_a.start()
            rdma_b.start()
            if h == 1:
                compute_halves(me, me, True)
            else:
                compute_halves(o_a, o_b, False)
            rdma_a.wait()
            rdma_b.wait()

        compute_halves(lax.rem(me + 1, N_DEV),
                       lax.rem(me + 3, N_DEV), False)

    out, _ = pl.pallas_call(
        body,
        out_shape=[
            jax.ShapeDtypeStruct((N_DEV * M_PER, N_SH), jnp.float32),
            jax.ShapeDtypeStruct((N_DEV, M_PER, K), jnp.bfloat16),
        ],
        in_specs=[
            pl.BlockSpec(memory_space=pl.ANY),
            pl.BlockSpec(memory_space=pltpu.VMEM),
        ],
        out_specs=[
            pl.BlockSpec(memory_space=pl.ANY),
            pl.BlockSpec(memory_space=pl.ANY),
        ],
        scratch_shapes=[
            pltpu.VMEM((TILE, K), jnp.bfloat16),
            pltpu.VMEM((TILE, N_SH), jnp.float32),
            pltpu.SemaphoreType.DMA((N_DEV - 1,)),
            pltpu.SemaphoreType.DMA((N_DEV - 1,)),
            pltpu.SemaphoreType.DMA((N_DEV - 1,)),
            pltpu.SemaphoreType.DMA((N_DEV - 1,)),
            pltpu.SemaphoreType.DMA,
            pltpu.SemaphoreType.DMA,
        ],
        compiler_params=pltpu.CompilerParams(collective_id=0),
    )(x, w)
    return out


# device time: 642914 ns/iter; 1.0534x vs baseline; 1.0534x over previous
import jax
import jax.numpy as jnp
from jax import lax
from jax.experimental import pallas as pl
from jax.experimental.pallas import tpu as pltpu

N_DEV = 4
M_PER = 2048
HALF = 1024
TILE = 512
K = 8192
N_SH = 1024
W_CAST_ROWS = 512


def _gelu(y):
    c = 0.7978845608028654
    return 0.5 * y * (1.0 + jnp.tanh(c * (y + 0.044715 * y * y * y)))


def kernel(x, w_mat):
    def body(x_ref, w_ref, out_ref, xg_ref, xtile, otile, cf32, w_vmem,
             send_a, recv_a, send_b, recv_b, in_sem, out_sem):
        me = lax.axis_index("i")
        left = lax.rem(me + N_DEV - 1, N_DEV)
        right = lax.rem(me + 1, N_DEV)

        barrier = pltpu.get_barrier_semaphore()
        for nbr in (left, right):
            pl.semaphore_signal(barrier, inc=1, device_id=(nbr,),
                                device_id_type=pl.DeviceIdType.MESH)
        pl.semaphore_wait(barrier, 2)

        def cast_x_tile(t):
            cp = pltpu.make_async_copy(
                x_ref.at[pl.ds(t * TILE, TILE), :], cf32, in_sem)
            cp.start()
            cp.wait()
            xtile[...] = cf32[...].astype(jnp.bfloat16)
            cp2 = pltpu.make_async_copy(
                xtile, xg_ref.at[me, pl.ds(t * TILE, TILE), :], out_sem)
            cp2.start()
            cp2.wait()

        def ring_rdma(origin, row0, rows, sems_idx, to_right):
            sl = (origin, pl.ds(row0, rows))
            return pltpu.make_async_remote_copy(
                src_ref=xg_ref.at[sl], dst_ref=xg_ref.at[sl],
                send_sem=(send_a if to_right else send_b).at[sems_idx],
                recv_sem=(recv_a if to_right else recv_b).at[sems_idx],
                device_id=(right if to_right else left,),
                device_id_type=pl.DeviceIdType.MESH)

        def compute_tiles(o_top, o_bot, start, step, n):
            def tile_body(i, carry):
                t = start + step * i
                origin = jnp.where(t < 2, o_top, o_bot)
                r = t * TILE
                cp_in = pltpu.make_async_copy(
                    xg_ref.at[origin, pl.ds(r, TILE), :], xtile, in_sem)
                cp_in.start()
                cp_in.wait()
                y = jnp.dot(xtile[...], w_vmem[...],
                            preferred_element_type=jnp.float32)
                otile[...] = _gelu(y)
                cp_out = pltpu.make_async_copy(
                    otile, out_ref.at[pl.ds(origin * M_PER + r, TILE), :],
                    out_sem)
                cp_out.start()
                cp_out.wait()
                return carry
            lax.fori_loop(0, n, tile_body, 0)

        cast_x_tile(0)
        cast_x_tile(1)
        rdma_a1 = ring_rdma(me, 0, HALF, 0, True)
        rdma_a1.start()
        cast_x_tile(2)
        cast_x_tile(3)
        rdma_b1 = ring_rdma(me, HALF, HALF, 0, False)
        rdma_b1.start()

        for i in range(K // W_CAST_ROWS):
            cp = pltpu.make_async_copy(
                w_ref.at[pl.ds(i * W_CAST_ROWS, W_CAST_ROWS), :], otile,
                in_sem)
            cp.start()
            cp.wait()
            w_vmem[pl.ds(i * W_CAST_ROWS, W_CAST_ROWS), :] = (
                otile[...].astype(jnp.bfloat16))

        compute_tiles(me, me, 0, 1, 4)
        rdma_a1.wait()
        rdma_b1.wait()

        o_a2 = lax.rem(me + 3, N_DEV)
        o_b2 = lax.rem(me + 1, N_DEV)
        rdma_a2 = ring_rdma(o_a2, 0, HALF, 1, True)
        rdma_b2 = ring_rdma(o_b2, HALF, HALF, 1, False)
        rdma_a2.start()
        rdma_b2.start()
        compute_tiles(o_a2, o_b2, 0, 1, 4)
        rdma_a2.wait()
        rdma_b2.wait()

        o3 = lax.rem(me + 2, N_DEV)
        a_q0 = ring_rdma(o3, 0, TILE, 2, True)
        a_q1 = ring_rdma(o3, TILE, TILE, 3, True)
        b_q0 = ring_rdma(o3, HALF, TILE, 2, False)
        b_q1 = ring_rdma(o3, HALF + TILE, TILE, 3, False)
        a_q0.start()
        a_q1.start()
        b_q0.start()
        b_q1.start()
        compute_tiles(o3, o3, 0, 1, 4)
        o_at = lax.rem(me + 1, N_DEV)
        o_bt = lax.rem(me + 3, N_DEV)
        a_q0.wait()
        b_q0.wait()
        compute_tiles(o_at, o_bt, 0, 2, 2)
        a_q1.wait()
        b_q1.wait()
        compute_tiles(o_at, o_bt, 1, 2, 2)

    out, _ = pl.pallas_call(
        body,
        out_shape=[
            jax.ShapeDtypeStruct((N_DEV * M_PER, N_SH), jnp.float32),
            jax.ShapeDtypeStruct((N_DEV, M_PER, K), jnp.bfloat16),
        ],
        in_specs=[
            pl.BlockSpec(memory_space=pl.ANY),
            pl.BlockSpec(memory_space=pl.ANY),
        ],
        out_specs=[
            pl.BlockSpec(memory_space=pl.ANY),
            pl.BlockSpec(memory_space=pl.ANY),
        ],
        scratch_shapes=[
            pltpu.VMEM((TILE, K), jnp.bfloat16),
            pltpu.VMEM((TILE, N_SH), jnp.float32),
            pltpu.VMEM((TILE, K), jnp.float32),
            pltpu.VMEM((K, N_SH), jnp.bfloat16),
            pltpu.SemaphoreType.DMA((4,)),
            pltpu.SemaphoreType.DMA((4,)),
            pltpu.SemaphoreType.DMA((4,)),
            pltpu.SemaphoreType.DMA((4,)),
            pltpu.SemaphoreType.DMA,
            pltpu.SemaphoreType.DMA,
        ],
        compiler_params=pltpu.CompilerParams(
            collective_id=0, vmem_limit_bytes=60 * 1024 * 1024),
    )(x, w_mat)
    return out


# device time: 570483 ns/iter; 1.1871x vs baseline; 1.1270x over previous
import jax
import jax.numpy as jnp
from jax import lax
from jax.experimental import pallas as pl
from jax.experimental.pallas import tpu as pltpu

N_DEV = 4
M_PER = 2048
K = 8192
N_SH = 1024
TILE = 512
WHALF = K // 2
WQ = K // 4
XC = 256


def _gelu(y):
    c = 0.7978845608028654
    return 0.5 * y * (1.0 + jnp.tanh(c * (y + 0.044715 * y * y * y)))


def kernel(x, w_mat):
    def body(x_ref, w_ref, out_ref, wg_ref, xbf_ref, res_ref,
             xtile, otile, cf32, wf32, wbf, w_vmem,
             send_a, recv_a, send_b, recv_b, send_res, recv_res,
             in_sem, out_sem, stage_sem):
        me = lax.axis_index("i")
        left = lax.rem(me + 3, N_DEV)
        right = lax.rem(me + 1, N_DEV)
        diag = lax.rem(me + 2, N_DEV)

        barrier = pltpu.get_barrier_semaphore()
        for nbr in (left, right):
            pl.semaphore_signal(barrier, inc=1, device_id=(nbr,),
                                device_id_type=pl.DeviceIdType.MESH)
        pl.semaphore_wait(barrier, 2)

        def cast_w_quarter(q):
            cp = pltpu.make_async_copy(
                w_ref.at[pl.ds(q * WQ, WQ), :], wf32, in_sem)
            cp.start()
            cp.wait()
            wbf[...] = wf32[...].astype(jnp.bfloat16)
            w_vmem[pl.ds(q * WQ, WQ), :] = wbf[...]
            cp2 = pltpu.make_async_copy(
                wbf, wg_ref.at[me, pl.ds(q * WQ, WQ), :], out_sem)
            cp2.start()
            cp2.wait()

        def ring_rdma(origin, row0, sems_idx, to_right):
            sl = (origin, pl.ds(row0, WHALF))
            return pltpu.make_async_remote_copy(
                src_ref=wg_ref.at[sl], dst_ref=wg_ref.at[sl],
                send_sem=(send_a if to_right else send_b).at[sems_idx],
                recv_sem=(recv_a if to_right else recv_b).at[sems_idx],
                device_id=(right if to_right else left,),
                device_id_type=pl.DeviceIdType.MESH)

        def own_tiles(lo, hi):
            def tile_body(t, carry):
                cp_in = pltpu.make_async_copy(
                    xbf_ref.at[pl.ds(t * TILE, TILE), :], xtile, in_sem)
                cp_in.start()
                cp_in.wait()
                y = jnp.dot(xtile[...], w_vmem[...],
                            preferred_element_type=jnp.float32)
                otile[...] = _gelu(y)
                cp_out = pltpu.make_async_copy(
                    otile, out_ref.at[pl.ds(me * M_PER + t * TILE, TILE), :],
                    out_sem)
                cp_out.start()
                cp_out.wait()
                return carry
            lax.fori_loop(lo, hi, tile_body, 0)

        def peer_block(peer, slot):
            cp_w = pltpu.make_async_copy(wg_ref.at[peer], w_vmem, stage_sem)
            cp_w.start()
            cp_w.wait()

            def tile_body(t, carry):
                cp_in = pltpu.make_async_copy(
                    xbf_ref.at[pl.ds(t * TILE, TILE), :], xtile, in_sem)
                cp_in.start()
                cp_in.wait()
                y = jnp.dot(xtile[...], w_vmem[...],
                            preferred_element_type=jnp.float32)
                otile[...] = _gelu(y)
                cp_out = pltpu.make_async_copy(
                    otile, res_ref.at[slot, pl.ds(t * TILE, TILE), :],
                    out_sem)
                cp_out.start()
                cp_out.wait()
                return carry
            lax.fori_loop(0, N_DEV, tile_body, 0)
            rdma = pltpu.make_async_remote_copy(
                src_ref=res_ref.at[slot],
                dst_ref=out_ref.at[pl.ds(me * M_PER, M_PER), :],
                send_sem=send_res.at[slot], recv_sem=recv_res.at[slot],
                device_id=(peer,), device_id_type=pl.DeviceIdType.MESH)
            rdma.start()

        cast_w_quarter(0)
        cast_w_quarter(1)
        a1 = ring_rdma(me, 0, 0, True)
        a1.start()
        cast_w_quarter(2)
        cast_w_quarter(3)
        b1 = ring_rdma(me, WHALF, 0, False)
        b1.start()

        for i in range(M_PER // XC):
            cp = pltpu.make_async_copy(
                x_ref.at[pl.ds(i * XC, XC), :], cf32, in_sem)
            cp.start()
            cp.wait()
            xtile[pl.ds(0, XC), :] = cf32[...].astype(jnp.bfloat16)
            cp2 = pltpu.make_async_copy(
                xtile.at[pl.ds(0, XC), :], xbf_ref.at[pl.ds(i * XC, XC), :],
                out_sem)
            cp2.start()
            cp2.wait()

        own_tiles(0, 2)
        a1.wait()
        b1.wait()

        a2 = ring_rdma(lax.rem(me + 3, N_DEV), 0, 1, True)
        b2 = ring_rdma(lax.rem(me + 1, N_DEV), WHALF, 1, False)
        a2.start()
        b2.start()
        own_tiles(2, 4)
        a2.wait()
        b2.wait()

        a3 = ring_rdma(diag, 0, 2, True)
        b3 = ring_rdma(diag, WHALF, 2, False)
        a3.start()
        b3.start()
        peer_block(diag, 1)
        a3.wait()
        b3.wait()

        peer_block(right, 0)
        peer_block(left, 2)

        for slot in range(3):
            d = pltpu.make_async_remote_copy(
                src_ref=res_ref.at[slot], dst_ref=res_ref.at[slot],
                send_sem=send_res.at[slot], recv_sem=recv_res.at[slot],
                device_id=(me,), device_id_type=pl.DeviceIdType.MESH)
            d.wait_send()
        for j, s in enumerate((left, diag, right)):
            d = pltpu.make_async_remote_copy(
                src_ref=res_ref.at[j],
                dst_ref=out_ref.at[pl.ds(s * M_PER, M_PER), :],
                send_sem=send_res.at[j], recv_sem=recv_res.at[j],
                device_id=(me,), device_id_type=pl.DeviceIdType.MESH)
            d.wait_recv()

    out = pl.pallas_call(
        body,
        out_shape=[
            jax.ShapeDtypeStruct((N_DEV * M_PER, N_SH), jnp.float32),
            jax.ShapeDtypeStruct((N_DEV, K, N_SH), jnp.bfloat16),
            jax.ShapeDtypeStruct((M_PER, K), jnp.bfloat16),
            jax.ShapeDtypeStruct((3, M_PER, N_SH), jnp.float32),
        ],
        in_specs=[
            pl.BlockSpec(memory_space=pl.ANY),
            pl.BlockSpec(memory_space=pl.ANY),
        ],
        out_specs=[pl.BlockSpec(memory_space=pl.ANY)] * 4,
        scratch_shapes=[
            pltpu.VMEM((TILE, K), jnp.bfloat16),
            pltpu.VMEM((TILE, N_SH), jnp.float32),
            pltpu.VMEM((XC, K), jnp.float32),
            pltpu.VMEM((WQ, N_SH), jnp.float32),
            pltpu.VMEM((WQ, N_SH), jnp.bfloat16),
            pltpu.VMEM((K, N_SH), jnp.bfloat16),
            pltpu.SemaphoreType.DMA((3,)),
            pltpu.SemaphoreType.DMA((3,)),
            pltpu.SemaphoreType.DMA((3,)),
            pltpu.SemaphoreType.DMA((3,)),
            pltpu.SemaphoreType.DMA((3,)),
            pltpu.SemaphoreType.DMA((3,)),
            pltpu.SemaphoreType.DMA,
            pltpu.SemaphoreType.DMA,
            pltpu.SemaphoreType.DMA,
        ],
        compiler_params=pltpu.CompilerParams(
            collective_id=0, vmem_limit_bytes=60 * 1024 * 1024),
    )(x, w_mat)
    return out[0]
